# baseline (speedup 1.0000x reference)
import functools

import jax
import jax.numpy as jnp
from jax.experimental import pallas as pl
from jax.experimental.pallas import tpu as pltpu

_NUM_USERS = 16384
_NUM_ITEMS = 4096
_TOP_K = 10
_L_M = 0.7


def _apply_norm_dense(A):
    n = A.shape[0]
    eye = jnp.eye(n, dtype=A.dtype)
    A = A * (1.0 - eye) + eye
    deg = jnp.sum((A != 0).astype(A.dtype), axis=1)
    dis = jnp.where(deg > 0, jax.lax.rsqrt(jnp.maximum(deg, 1e-12)), 0.0)
    return dis[:, None] * A * dis[None, :]


def _knn_dense_adj(feats):
    f = feats / (jnp.linalg.norm(feats, axis=-1, keepdims=True) + 1e-12)
    sim = f @ f.T
    knn_val, knn_ind = jax.lax.top_k(sim, _TOP_K)
    n = feats.shape[0]
    rows = jnp.repeat(jnp.arange(n), _TOP_K)
    cols = knn_ind.reshape(-1)
    vals = knn_val.reshape(-1)
    A = jnp.zeros((n, n), feats.dtype).at[rows, cols].set(vals)
    return _apply_norm_dense(A)


def _combine_body(e0_ref, s1_ref, s2_ref, h_ref, gu_ref, gi_ref):
    blk = pl.program_id(0)
    mean = (e0_ref[...] + s1_ref[...] + s2_ref[...]) * (1.0 / 3.0)

    @pl.when(blk < _NUM_USERS // 2048)
    def _():
        gu_ref[...] = mean

    @pl.when(blk >= _NUM_USERS // 2048)
    def _():
        h = h_ref[...]
        nrm = jnp.sqrt(jnp.sum(h * h, axis=-1, keepdims=True))
        gi_ref[...] = mean + h / (nrm + 1e-12)


def _combine(e0, s1, s2, h):
    n_user_blocks = _NUM_USERS // 2048
    n_item_blocks = _NUM_ITEMS // 2048
    grid = n_user_blocks + n_item_blocks
    k = 64

    def in_spec_full(i):
        return (i, 0)

    def h_spec(i):
        return (jnp.maximum(i - n_user_blocks, 0), 0)

    def gu_spec(i):
        return (jnp.minimum(i, n_user_blocks - 1), 0)

    def gi_spec(i):
        return (jnp.maximum(i - n_user_blocks, 0), 0)

    gu, gi = pl.pallas_call(
        _combine_body,
        grid=(grid,),
        in_specs=[
            pl.BlockSpec((2048, k), in_spec_full),
            pl.BlockSpec((2048, k), in_spec_full),
            pl.BlockSpec((2048, k), in_spec_full),
            pl.BlockSpec((2048, k), h_spec),
        ],
        out_specs=[
            pl.BlockSpec((2048, k), gu_spec),
            pl.BlockSpec((2048, k), gi_spec),
        ],
        out_shape=[
            jax.ShapeDtypeStruct((_NUM_USERS, k), jnp.float32),
            jax.ShapeDtypeStruct((_NUM_ITEMS, k), jnp.float32),
        ],
    )(e0, s1, s2, h)
    return gu, gi


def kernel(Gu, Gi, feat_visual, feat_textual, W_visual, b_visual,
           W_textual, b_textual, w_imp, ui_edge_index, ui_values):
    mods = ['visual', 'textual']
    feats = {'visual': feat_visual, 'textual': feat_textual}
    proj = {'visual': (W_visual, b_visual), 'textual': (W_textual, b_textual)}
    Gim = {m: feats[m] / (jnp.linalg.norm(feats[m], axis=-1, keepdims=True) + 1e-12) for m in mods}
    Sim = {m: _knn_dense_adj(feats[m]) for m in mods}
    projected = {m: Gim[m] @ proj[m][0] + proj[m][1] for m in mods}
    sw = jax.nn.softmax(w_imp, axis=0)
    learned_parts = []
    original_parts = []
    for m_id, m in enumerate(mods):
        Am = _knn_dense_adj(projected[m])
        learned_parts.append(sw[m_id] * Am)
        original_parts.append(sw[m_id] * Sim[m])
    learned_adj = _apply_norm_dense(learned_parts[0] + learned_parts[1])
    original_adj = _apply_norm_dense(original_parts[0] + original_parts[1])
    Si = (1.0 - _L_M) * learned_adj + _L_M * original_adj
    h = Si @ Gi

    ego = jnp.concatenate([Gu, Gi], axis=0)
    row = ui_edge_index[0]
    col = ui_edge_index[1]
    e = ego
    embs = [ego]
    for _ in range(2):
        msg = ui_values[:, None] * e[col]
        e = jax.ops.segment_sum(msg, row, num_segments=_NUM_USERS + _NUM_ITEMS)
        embs.append(e)
    return _combine(embs[0], embs[1], embs[2], h)


# speedup vs baseline: 1.5279x; 1.5279x over previous
import functools

import jax
import jax.numpy as jnp
import numpy as np
from jax.experimental import pallas as pl
from jax.experimental.pallas import tpu as pltpu

_NUM_USERS = 16384
_NUM_ITEMS = 4096
_TOP_K = 10
_L_M = 0.7
_NEG = -3.4e38

_INTERPRET = False


def _rownorm_body(f_ref, o_ref):
    f = f_ref[...]
    nrm = jnp.sqrt(jnp.sum(f * f, axis=-1, keepdims=True))
    o_ref[...] = f / (nrm + 1e-12)


def _rownorm(f, block_rows=512):
    n, d = f.shape
    return pl.pallas_call(
        _rownorm_body,
        grid=(n // block_rows,),
        in_specs=[pl.BlockSpec((block_rows, d), lambda i: (i, 0))],
        out_specs=pl.BlockSpec((block_rows, d), lambda i: (i, 0)),
        out_shape=jax.ShapeDtypeStruct((n, d), jnp.float32),
        interpret=_INTERPRET,
    )(f)


def _proj_body(f_ref, w_ref, b_ref, o_ref):
    p = jax.lax.dot_general(f_ref[...], w_ref[...], (((1,), (0,)), ((), ())),
                            preferred_element_type=jnp.float32)
    p = p + b_ref[...]
    nrm = jnp.sqrt(jnp.sum(p * p, axis=-1, keepdims=True))
    o_ref[...] = p / (nrm + 1e-12)


def _proj_norm(f, w, b, block_rows=1024):
    n, d = f.shape
    k = w.shape[1]
    return pl.pallas_call(
        _proj_body,
        grid=(n // block_rows,),
        in_specs=[
            pl.BlockSpec((block_rows, d), lambda i: (i, 0)),
            pl.BlockSpec((d, k), lambda i: (0, 0)),
            pl.BlockSpec((1, k), lambda i: (0, 0)),
        ],
        out_specs=pl.BlockSpec((block_rows, k), lambda i: (i, 0)),
        out_shape=jax.ShapeDtypeStruct((n, k), jnp.float32),
        interpret=_INTERPRET,
    )(f, w, b.reshape(1, k))


def _simtopk_body(fi_ref, fj_ref, vals_ref, inds_ref, *, cblk, nj, k):
    j = pl.program_id(1)

    @pl.when(j == 0)
    def _():
        vals_ref[...] = jnp.full(vals_ref.shape, _NEG, jnp.float32)
        inds_ref[...] = jnp.zeros(inds_ref.shape, jnp.int32)

    scores = jax.lax.dot_general(
        fi_ref[...], fj_ref[...], (((1,), (1,)), ((), ())),
        preferred_element_type=jnp.float32)
    r = scores.shape[0]
    width = k + cblk
    col_iota = jax.lax.broadcasted_iota(jnp.int32, (r, cblk), 1)
    comb_v = jnp.concatenate([vals_ref[...], scores], axis=1)
    comb_i = jnp.concatenate([inds_ref[...], j * cblk + col_iota], axis=1)
    pos_iota = jax.lax.broadcasted_iota(jnp.int32, (r, width), 1)
    new_v = []
    new_i = []
    for _ in range(k):
        m = jnp.max(comb_v, axis=1, keepdims=True)
        hit = comb_v == m
        pos = jnp.min(jnp.where(hit, pos_iota, jnp.int32(2 ** 30)),
                      axis=1, keepdims=True)
        sel = pos_iota == pos
        gidx = jnp.sum(jnp.where(sel, comb_i, 0), axis=1, keepdims=True)
        new_v.append(m)
        new_i.append(gidx)
        comb_v = jnp.where(sel, _NEG, comb_v)
    vals_ref[...] = jnp.concatenate(new_v, axis=1)
    inds_ref[...] = jnp.concatenate(new_i, axis=1)


def _sim_topk(f, rblk=256, cblk=512, k=_TOP_K):
    n, d = f.shape
    ni, nj = n // rblk, n // cblk
    body = functools.partial(_simtopk_body, cblk=cblk, nj=nj, k=k)
    vals, inds = pl.pallas_call(
        body,
        grid=(ni, nj),
        in_specs=[
            pl.BlockSpec((rblk, d), lambda i, j: (i, 0)),
            pl.BlockSpec((cblk, d), lambda i, j: (j, 0)),
        ],
        out_specs=[
            pl.BlockSpec((rblk, k), lambda i, j: (i, 0)),
            pl.BlockSpec((rblk, k), lambda i, j: (i, 0)),
        ],
        out_shape=[
            jax.ShapeDtypeStruct((n, k), jnp.float32),
            jax.ShapeDtypeStruct((n, k), jnp.int32),
        ],
        interpret=_INTERPRET,
    )(f, f)
    return vals, inds


def _prep_body(wimp_ref, gi_ref,
               ivL_ref, vvL_ref, itL_ref, vtL_ref,
               ivR_ref, vvR_ref, itR_ref, vtR_ref,
               cidx_ref, cw_ref, tab_ref, hdiag_ref, *, n, rblk, k):
    wimp = wimp_ref[0, 0]
    wimp2 = wimp_ref[0, 1]
    mx = jnp.maximum(wimp, wimp2)
    e1 = jnp.exp(wimp - mx)
    e2 = jnp.exp(wimp2 - mx)
    sw0 = e1 / (e1 + e2)
    sw1 = e2 / (e1 + e2)

    row = (pl.program_id(0) * rblk
           + jax.lax.broadcasted_iota(jnp.int32, (rblk, 1), 0))

    def graph_deg(idx, val):
        valid = (idx != row) & (val != 0.0)
        return 1.0 + jnp.sum(valid.astype(jnp.float32), axis=1, keepdims=True)

    ivL, vvL = ivL_ref[...], vvL_ref[...]
    itL, vtL = itL_ref[...], vtL_ref[...]
    ivR, vvR = ivR_ref[...], vvR_ref[...]
    itR, vtR = itR_ref[...], vtR_ref[...]

    degLv = graph_deg(ivL, vvL)
    degLt = graph_deg(itL, vtL)
    degRv = graph_deg(ivR, vvR)
    degRt = graph_deg(itR, vtR)
    disLv = jax.lax.rsqrt(degLv)
    disLt = jax.lax.rsqrt(degLt)
    disRv = jax.lax.rsqrt(degRv)
    disRt = jax.lax.rsqrt(degRt)

    def union_deg(idx_a, val_a, idx_b, val_b):
        idx = jnp.concatenate([idx_a, idx_b], axis=1)
        val = jnp.concatenate([val_a, val_b], axis=1)
        valid = (idx != row) & (val != 0.0)
        cnt = jnp.zeros((rblk, 1), jnp.float32)
        for c in range(2 * k):
            ic = idx[:, c:c + 1]
            vc = valid[:, c:c + 1]
            dup = jnp.zeros((rblk, 1), jnp.bool_)
            for cc in range(c):
                dup = dup | (valid[:, cc:cc + 1] & (idx[:, cc:cc + 1] == ic))
            cnt = cnt + jnp.where(vc & ~dup, 1.0, 0.0)
        return 1.0 + cnt

    degL = union_deg(ivL, vvL, itL, vtL)
    degR = union_deg(ivR, vvR, itR, vtR)
    disL = jax.lax.rsqrt(degL)
    disR = jax.lax.rsqrt(degR)

    wL = 1.0 - _L_M
    wR = _L_M

    def group_w(outer_dis, outer_w, sw, inner_dis, idx, val):
        return outer_w * outer_dis * sw * inner_dis * val * (idx != row)

    cw_ref[...] = jnp.concatenate([
        group_w(disL, wL, sw0, disLv, ivL, vvL),
        group_w(disL, wL, sw1, disLt, itL, vtL),
        group_w(disR, wR, sw0, disRv, ivR, vvR),
        group_w(disR, wR, sw1, disRt, itR, vtR),
    ], axis=1)
    cidx_ref[...] = jnp.concatenate(
        [ivL, itL + n, ivR + 2 * n, itR + 3 * n], axis=1)

    gi = gi_ref[...]
    tab_ref[0, :, :] = (disLv * disL) * gi
    tab_ref[1, :, :] = (disLt * disL) * gi
    tab_ref[2, :, :] = (disRv * disR) * gi
    tab_ref[3, :, :] = (disRt * disR) * gi
    hdiag_ref[...] = (wL / degL + wR / degR) * gi


def _prep(w_imp, gi, ivL, vvL, itL, vtL, ivR, vvR, itR, vtR, rblk=512):
    n, k = ivL.shape
    body = functools.partial(_prep_body, n=n, rblk=rblk, k=k)
    cidx, cw, tab, hdiag = pl.pallas_call(
        body,
        grid=(n // rblk,),
        in_specs=[
            pl.BlockSpec((1, 2), lambda i: (0, 0)),
            pl.BlockSpec((rblk, 64), lambda i: (i, 0)),
        ] + [pl.BlockSpec((rblk, k), lambda i: (i, 0))] * 8,
        out_specs=[
            pl.BlockSpec((rblk, 4 * k), lambda i: (i, 0)),
            pl.BlockSpec((rblk, 4 * k), lambda i: (i, 0)),
            pl.BlockSpec((4, rblk, 64), lambda i: (0, i, 0)),
            pl.BlockSpec((rblk, 64), lambda i: (i, 0)),
        ],
        out_shape=[
            jax.ShapeDtypeStruct((n, 4 * k), jnp.int32),
            jax.ShapeDtypeStruct((n, 4 * k), jnp.float32),
            jax.ShapeDtypeStruct((4, n, 64), jnp.float32),
            jax.ShapeDtypeStruct((n, 64), jnp.float32),
        ],
        interpret=_INTERPRET,
    )(w_imp.reshape(1, 2), gi, ivL, vvL, itL, vtL, ivR, vvR, itR, vtR)
    return cidx, cw, tab.reshape(4 * n, 64), hdiag


def _combine_body(e0_ref, s1_ref, s2_ref, hg_ref, hd_ref, gu_ref, gi_ref,
                  *, n_user_blocks):
    blk = pl.program_id(0)
    mean = (e0_ref[...] + s1_ref[...] + s2_ref[...]) * (1.0 / 3.0)

    @pl.when(blk < n_user_blocks)
    def _():
        gu_ref[...] = mean

    @pl.when(blk >= n_user_blocks)
    def _():
        h = hg_ref[...] + hd_ref[...]
        nrm = jnp.sqrt(jnp.sum(h * h, axis=-1, keepdims=True))
        gi_ref[...] = mean + h / (nrm + 1e-12)


def _combine(e0, s1, s2, hg, hd):
    blk = 2048
    nub = _NUM_USERS // blk
    nib = _NUM_ITEMS // blk
    kdim = 64

    body = functools.partial(_combine_body, n_user_blocks=nub)
    gu, gi = pl.pallas_call(
        body,
        grid=(nub + nib,),
        in_specs=[
            pl.BlockSpec((blk, kdim), lambda i: (i, 0)),
            pl.BlockSpec((blk, kdim), lambda i: (i, 0)),
            pl.BlockSpec((blk, kdim), lambda i: (i, 0)),
            pl.BlockSpec((blk, kdim), lambda i: (jnp.maximum(i - nub, 0), 0)),
            pl.BlockSpec((blk, kdim), lambda i: (jnp.maximum(i - nub, 0), 0)),
        ],
        out_specs=[
            pl.BlockSpec((blk, kdim), lambda i: (jnp.minimum(i, nub - 1), 0)),
            pl.BlockSpec((blk, kdim), lambda i: (jnp.maximum(i - nub, 0), 0)),
        ],
        out_shape=[
            jax.ShapeDtypeStruct((_NUM_USERS, kdim), jnp.float32),
            jax.ShapeDtypeStruct((_NUM_ITEMS, kdim), jnp.float32),
        ],
        interpret=_INTERPRET,
    )(e0, s1, s2, hg, hd)
    return gu, gi


def kernel(Gu, Gi, feat_visual, feat_textual, W_visual, b_visual,
           W_textual, b_textual, w_imp, ui_edge_index, ui_values):
    fv = _rownorm(feat_visual)
    ft = _rownorm(feat_textual)
    pv = _proj_norm(fv, W_visual, b_visual)
    pt = _proj_norm(ft, W_textual, b_textual)

    vvR, ivR = _sim_topk(fv)
    vtR, itR = _sim_topk(ft)
    vvL, ivL = _sim_topk(pv)
    vtL, itL = _sim_topk(pt)

    cidx, cw, tab, hdiag = _prep(w_imp, Gi, ivL, vvL, itL, vtL,
                                 ivR, vvR, itR, vtR)

    hg = jnp.sum(cw[:, :, None] * tab[cidx], axis=1)

    ego = jnp.concatenate([Gu, Gi], axis=0)
    row = ui_edge_index[0].astype(jnp.int32)
    col = ui_edge_index[1].astype(jnp.int32)
    e = ego
    embs = [ego]
    for _ in range(2):
        msg = ui_values[:, None] * e[col]
        e = jax.ops.segment_sum(msg, row, num_segments=_NUM_USERS + _NUM_ITEMS)
        embs.append(e)

    return _combine(embs[0], embs[1], embs[2], hg, hdiag)


# speedup vs baseline: 5.9831x; 3.9159x over previous
import functools

import jax
import jax.numpy as jnp
import numpy as np
from jax import lax
from jax.experimental import pallas as pl
from jax.experimental.pallas import tpu as pltpu
from jax.experimental.pallas import tpu_sc as plsc

_NUM_USERS = 16384
_NUM_ITEMS = 4096
_TOP_K = 10
_L_M = 0.7
_NEG = -3.4e38

_INTERPRET = False


def _rownorm_body(f_ref, o_ref):
    f = f_ref[...]
    nrm = jnp.sqrt(jnp.sum(f * f, axis=-1, keepdims=True))
    o_ref[...] = f / (nrm + 1e-12)


def _rownorm(f, block_rows=512):
    n, d = f.shape
    return pl.pallas_call(
        _rownorm_body,
        grid=(n // block_rows,),
        in_specs=[pl.BlockSpec((block_rows, d), lambda i: (i, 0))],
        out_specs=pl.BlockSpec((block_rows, d), lambda i: (i, 0)),
        out_shape=jax.ShapeDtypeStruct((n, d), jnp.float32),
        interpret=_INTERPRET,
    )(f)


def _proj_body(f_ref, w_ref, b_ref, o_ref):
    p = jax.lax.dot_general(f_ref[...], w_ref[...], (((1,), (0,)), ((), ())),
                            preferred_element_type=jnp.float32)
    p = p + b_ref[...]
    nrm = jnp.sqrt(jnp.sum(p * p, axis=-1, keepdims=True))
    o_ref[...] = p / (nrm + 1e-12)


def _proj_norm(f, w, b, block_rows=1024):
    n, d = f.shape
    k = w.shape[1]
    return pl.pallas_call(
        _proj_body,
        grid=(n // block_rows,),
        in_specs=[
            pl.BlockSpec((block_rows, d), lambda i: (i, 0)),
            pl.BlockSpec((d, k), lambda i: (0, 0)),
            pl.BlockSpec((1, k), lambda i: (0, 0)),
        ],
        out_specs=pl.BlockSpec((block_rows, k), lambda i: (i, 0)),
        out_shape=jax.ShapeDtypeStruct((n, k), jnp.float32),
        interpret=_INTERPRET,
    )(f, w, b.reshape(1, k))


def _simtopk_body(fi_ref, fj_ref, vals_ref, inds_ref, *, cblk, nj, k):
    j = pl.program_id(1)

    @pl.when(j == 0)
    def _():
        vals_ref[...] = jnp.full(vals_ref.shape, _NEG, jnp.float32)
        inds_ref[...] = jnp.zeros(inds_ref.shape, jnp.int32)

    scores = jax.lax.dot_general(
        fi_ref[...], fj_ref[...], (((1,), (1,)), ((), ())),
        preferred_element_type=jnp.float32)
    r = scores.shape[0]
    width = k + cblk
    col_iota = jax.lax.broadcasted_iota(jnp.int32, (r, cblk), 1)
    comb_v = jnp.concatenate([vals_ref[...], scores], axis=1)
    comb_i = jnp.concatenate([inds_ref[...], j * cblk + col_iota], axis=1)
    pos_iota = jax.lax.broadcasted_iota(jnp.int32, (r, width), 1)
    new_v = []
    new_i = []
    for _ in range(k):
        m = jnp.max(comb_v, axis=1, keepdims=True)
        hit = comb_v == m
        pos = jnp.min(jnp.where(hit, pos_iota, jnp.int32(2 ** 30)),
                      axis=1, keepdims=True)
        sel = pos_iota == pos
        gidx = jnp.sum(jnp.where(sel, comb_i, 0), axis=1, keepdims=True)
        new_v.append(m)
        new_i.append(gidx)
        comb_v = jnp.where(sel, _NEG, comb_v)
    vals_ref[...] = jnp.concatenate(new_v, axis=1)
    inds_ref[...] = jnp.concatenate(new_i, axis=1)


def _sim_topk(f, rblk=256, cblk=512, k=_TOP_K):
    n, d = f.shape
    ni, nj = n // rblk, n // cblk
    body = functools.partial(_simtopk_body, cblk=cblk, nj=nj, k=k)
    vals, inds = pl.pallas_call(
        body,
        grid=(ni, nj),
        in_specs=[
            pl.BlockSpec((rblk, d), lambda i, j: (i, 0)),
            pl.BlockSpec((cblk, d), lambda i, j: (j, 0)),
        ],
        out_specs=[
            pl.BlockSpec((rblk, k), lambda i, j: (i, 0)),
            pl.BlockSpec((rblk, k), lambda i, j: (i, 0)),
        ],
        out_shape=[
            jax.ShapeDtypeStruct((n, k), jnp.float32),
            jax.ShapeDtypeStruct((n, k), jnp.int32),
        ],
        interpret=_INTERPRET,
    )(f, f)
    return vals, inds


def _prep_body(wimp_ref, gi_ref,
               ivL_ref, vvL_ref, itL_ref, vtL_ref,
               ivR_ref, vvR_ref, itR_ref, vtR_ref,
               cidx_ref, cw_ref, tab_ref, hdiag_ref, *, n, rblk, k):
    wimp = wimp_ref[0, 0]
    wimp2 = wimp_ref[0, 1]
    mx = jnp.maximum(wimp, wimp2)
    e1 = jnp.exp(wimp - mx)
    e2 = jnp.exp(wimp2 - mx)
    sw0 = e1 / (e1 + e2)
    sw1 = e2 / (e1 + e2)

    row = (pl.program_id(0) * rblk
           + jax.lax.broadcasted_iota(jnp.int32, (rblk, 1), 0))

    def graph_deg(idx, val):
        valid = (idx != row) & (val != 0.0)
        return 1.0 + jnp.sum(valid.astype(jnp.float32), axis=1, keepdims=True)

    ivL, vvL = ivL_ref[...], vvL_ref[...]
    itL, vtL = itL_ref[...], vtL_ref[...]
    ivR, vvR = ivR_ref[...], vvR_ref[...]
    itR, vtR = itR_ref[...], vtR_ref[...]

    degLv = graph_deg(ivL, vvL)
    degLt = graph_deg(itL, vtL)
    degRv = graph_deg(ivR, vvR)
    degRt = graph_deg(itR, vtR)
    disLv = jax.lax.rsqrt(degLv)
    disLt = jax.lax.rsqrt(degLt)
    disRv = jax.lax.rsqrt(degRv)
    disRt = jax.lax.rsqrt(degRt)

    def union_deg(idx_a, val_a, idx_b, val_b):
        idx = jnp.concatenate([idx_a, idx_b], axis=1)
        val = jnp.concatenate([val_a, val_b], axis=1)
        valid = (idx != row) & (val != 0.0)
        cnt = jnp.zeros((rblk, 1), jnp.float32)
        for c in range(2 * k):
            ic = idx[:, c:c + 1]
            vc = valid[:, c:c + 1]
            dup = jnp.zeros((rblk, 1), jnp.bool_)
            for cc in range(c):
                dup = dup | (valid[:, cc:cc + 1] & (idx[:, cc:cc + 1] == ic))
            cnt = cnt + jnp.where(vc & ~dup, 1.0, 0.0)
        return 1.0 + cnt

    degL = union_deg(ivL, vvL, itL, vtL)
    degR = union_deg(ivR, vvR, itR, vtR)
    disL = jax.lax.rsqrt(degL)
    disR = jax.lax.rsqrt(degR)

    wL = 1.0 - _L_M
    wR = _L_M

    def group_w(outer_dis, outer_w, sw, inner_dis, idx, val):
        return outer_w * outer_dis * sw * inner_dis * val * (idx != row)

    cw_ref[...] = jnp.concatenate([
        group_w(disL, wL, sw0, disLv, ivL, vvL),
        group_w(disL, wL, sw1, disLt, itL, vtL),
        group_w(disR, wR, sw0, disRv, ivR, vvR),
        group_w(disR, wR, sw1, disRt, itR, vtR),
    ], axis=1)
    cidx_ref[...] = jnp.concatenate(
        [ivL, itL + n, ivR + 2 * n, itR + 3 * n], axis=1)

    gi = gi_ref[...]
    gi2 = jnp.concatenate([gi, gi], axis=1)
    tab_ref[0, :, :] = (disLv * disL) * gi2
    tab_ref[1, :, :] = (disLt * disL) * gi2
    tab_ref[2, :, :] = (disRv * disR) * gi2
    tab_ref[3, :, :] = (disRt * disR) * gi2
    hdiag_ref[...] = (wL / degL + wR / degR) * gi


def _prep(w_imp, gi, ivL, vvL, itL, vtL, ivR, vvR, itR, vtR, rblk=512):
    n, k = ivL.shape
    body = functools.partial(_prep_body, n=n, rblk=rblk, k=k)
    cidx, cw, tab, hdiag = pl.pallas_call(
        body,
        grid=(n // rblk,),
        in_specs=[
            pl.BlockSpec((1, 2), lambda i: (0, 0)),
            pl.BlockSpec((rblk, 64), lambda i: (i, 0)),
        ] + [pl.BlockSpec((rblk, k), lambda i: (i, 0))] * 8,
        out_specs=[
            pl.BlockSpec((rblk, 4 * k), lambda i: (i, 0)),
            pl.BlockSpec((rblk, 4 * k), lambda i: (i, 0)),
            pl.BlockSpec((4, rblk, 128), lambda i: (0, i, 0)),
            pl.BlockSpec((rblk, 64), lambda i: (i, 0)),
        ],
        out_shape=[
            jax.ShapeDtypeStruct((n, 4 * k), jnp.int32),
            jax.ShapeDtypeStruct((n, 4 * k), jnp.float32),
            jax.ShapeDtypeStruct((4, n, 128), jnp.float32),
            jax.ShapeDtypeStruct((n, 64), jnp.float32),
        ],
        interpret=_INTERPRET,
    )(w_imp.reshape(1, 2), gi, ivL, vvL, itL, vtL, ivR, vvR, itR, vtR)
    return cidx, cw, tab.reshape(4 * n, 128), hdiag


def _combine_body(e0_ref, s1_ref, s2_ref, hg_ref, hd_ref, gu_ref, gi_ref,
                  *, n_user_blocks):
    blk = pl.program_id(0)
    mean = (e0_ref[...] + s1_ref[...] + s2_ref[...]) * (1.0 / 3.0)

    @pl.when(blk < n_user_blocks)
    def _():
        gu_ref[...] = mean

    @pl.when(blk >= n_user_blocks)
    def _():
        h = hg_ref[...] + hd_ref[...]
        nrm = jnp.sqrt(jnp.sum(h * h, axis=-1, keepdims=True))
        gi_ref[...] = mean + h / (nrm + 1e-12)


def _combine(e0, s1, s2, hg, hd):
    blk = 2048
    nub = _NUM_USERS // blk
    nib = _NUM_ITEMS // blk
    kdim = 64

    body = functools.partial(_combine_body, n_user_blocks=nub)
    gu, gi = pl.pallas_call(
        body,
        grid=(nub + nib,),
        in_specs=[
            pl.BlockSpec((blk, kdim), lambda i: (i, 0)),
            pl.BlockSpec((blk, kdim), lambda i: (i, 0)),
            pl.BlockSpec((blk, kdim), lambda i: (i, 0)),
            pl.BlockSpec((blk, kdim), lambda i: (jnp.maximum(i - nub, 0), 0)),
            pl.BlockSpec((blk, kdim), lambda i: (jnp.maximum(i - nub, 0), 0)),
        ],
        out_specs=[
            pl.BlockSpec((blk, kdim), lambda i: (jnp.minimum(i, nub - 1), 0)),
            pl.BlockSpec((blk, kdim), lambda i: (jnp.maximum(i - nub, 0), 0)),
        ],
        out_shape=[
            jax.ShapeDtypeStruct((_NUM_USERS, kdim), jnp.float32),
            jax.ShapeDtypeStruct((_NUM_ITEMS, kdim), jnp.float32),
        ],
        interpret=_INTERPRET,
    )(e0, s1, s2, hg, hd)
    return gu, gi


_EPC = (1048576 // 2) // 16
_SCHUNK = 256
_NV = _SCHUNK // 128
_UPAIR = _NUM_USERS // 2
_IPAIR = _NUM_ITEMS // 2


def _ui_layer_body(emb_hbm, rowp2d_hbm, col2d_hbm, val_hbm, par_hbm,
                   zero_hbm, out_hbm, acc, rowb, colb, valb, parb, gbuf,
                   sem):
    c = lax.axis_index("c")
    s = lax.axis_index("s")
    npair = jnp.where(c == 0, _UPAIR, _IPAIR)
    zrows = npair // 16
    step = _IPAIR // 16
    sz = pl.multiple_of(s * zrows, step)
    pltpu.sync_copy(zero_hbm.at[pl.ds(sz, step)], acc.at[pl.ds(sz, step)])

    @pl.when(c == 0)
    def _():
        zrows_u = _UPAIR // 16
        for t in range(_UPAIR // _IPAIR - 1):
            off = (t + 1) * step
            b = pl.multiple_of(s * zrows_u + off, step)
            pltpu.sync_copy(zero_hbm.at[pl.ds(b, step)],
                            acc.at[pl.ds(b, step)])

    plsc.subcore_barrier()

    vec_base = (c * 16 + s) * (_EPC // 128)

    def chunk_body(g, carry):
        vb = pl.multiple_of(vec_base + g * _NV, _NV)
        pltpu.sync_copy(rowp2d_hbm.at[pl.ds(vb, _NV)], rowb)
        pltpu.sync_copy(col2d_hbm.at[pl.ds(vb, _NV)], colb)
        pltpu.sync_copy(val_hbm.at[pl.ds(vb * 128, _SCHUNK)], valb)
        pltpu.sync_copy(par_hbm.at[pl.ds(vb * 128, _SCHUNK)], parb)
        cps = [
            pltpu.async_copy(emb_hbm.at[colb.at[j]],
                             gbuf.at[pl.ds(j * 128, 128)], sem)
            for j in range(_NV)
        ]
        for cp in cps:
            cp.wait()

        def mul_body(g2, _):
            v16 = valb[pl.ds(g2 * 16, 16)]
            p16 = parb[pl.ds(g2 * 16, 16)]
            for l in range(16):
                w = v16[l]
                p = p16[l]
                w1 = w * p
                w0 = w - w1
                e = g2 * 16 + l
                for q in range(4):
                    gbuf[e, pl.ds(q * 16, 16)] = (
                        gbuf[e, pl.ds(q * 16, 16)] * w0)
                    gbuf[e, pl.ds(64 + q * 16, 16)] = (
                        gbuf[e, pl.ds(64 + q * 16, 16)] * w1)
            return _

        lax.fori_loop(0, _SCHUNK // 16, mul_body, None)
        for j in range(_NV):
            pltpu.sync_copy(gbuf.at[pl.ds(j * 128, 128)],
                            acc.at[rowb.at[j]], add=True)
        return carry

    lax.fori_loop(0, _EPC // _SCHUNK, chunk_body, None)
    plsc.subcore_barrier()

    so = pl.multiple_of(s * zrows, step)
    obase = pl.multiple_of(jnp.where(c == 0, 0, _UPAIR) + s * zrows, step)
    pltpu.sync_copy(acc.at[pl.ds(so, step)], out_hbm.at[pl.ds(obase, step)])

    @pl.when(c == 0)
    def _():
        zrows_u = _UPAIR // 16
        for t in range(_UPAIR // _IPAIR - 1):
            off = (t + 1) * step
            b = pl.multiple_of(s * zrows_u + off, step)
            pltpu.sync_copy(acc.at[pl.ds(b, step)],
                            out_hbm.at[pl.ds(b, step)])


def _ui_layer(emb_dup, rowp2d, col2d, val, par, zero_tab):
    mesh = plsc.VectorSubcoreMesh(core_axis_name="c", subcore_axis_name="s")
    f = functools.partial(
        pl.kernel,
        mesh=mesh,
        out_type=jax.ShapeDtypeStruct((_UPAIR + _IPAIR, 128), jnp.float32),
        scratch_types=[
            pltpu.VMEM_SHARED((_UPAIR, 128), jnp.float32),
            pltpu.VMEM((_NV, 128), jnp.int32),
            pltpu.VMEM((_NV, 128), jnp.int32),
            pltpu.VMEM((_SCHUNK,), jnp.float32),
            pltpu.VMEM((_SCHUNK,), jnp.float32),
            pltpu.VMEM((_SCHUNK, 128), jnp.float32),
            pltpu.SemaphoreType.DMA,
        ],
    )(_ui_layer_body)
    return f(emb_dup, rowp2d, col2d, val, par, zero_tab)


def _hgather_body(tab_hbm, cidx2d_hbm, cw_hbm, out_hbm,
                  idxb, wb, gbuf, ob, sem):
    c = lax.axis_index("c")
    s = lax.axis_index("s")
    t = c * 16 + s
    pltpu.sync_copy(cidx2d_hbm.at[pl.ds(pl.multiple_of(t * 40, 40), 40)],
                    idxb)
    pltpu.sync_copy(cw_hbm.at[pl.ds(pl.multiple_of(t * 5120, 5120), 5120)],
                    wb.at[pl.ds(0, 5120)])
    for g in range(8):
        cps = [
            pltpu.async_copy(tab_hbm.at[idxb.at[g * 5 + j]],
                             gbuf.at[pl.ds(j * 128, 128)], sem)
            for j in range(5)
        ]
        for cp in cps:
            cp.wait()

        def row_body(r, _):
            zero = jnp.zeros((16,), jnp.float32)
            accs = [zero] * 4
            wbase = g * 640 + r * 40
            for blk in range(3):
                v16 = wb[pl.ds(wbase + blk * 16, 16)]
                nl = 16 if blk < 2 else 8
                for l in range(nl):
                    w = v16[l]
                    fi = r * 40 + blk * 16 + l
                    for q in range(4):
                        accs[q] = accs[q] + w * gbuf[fi, pl.ds(q * 16, 16)]
            for q in range(4):
                ob[g * 16 + r, pl.ds(q * 16, 16)] = accs[q]
            return _

        lax.fori_loop(0, 16, row_body, None)
    pltpu.sync_copy(ob, out_hbm.at[pl.ds(pl.multiple_of(t * 128, 128), 128)])


def _hgather(tab, cidx2d, cw_flat):
    mesh = plsc.VectorSubcoreMesh(core_axis_name="c", subcore_axis_name="s")
    f = functools.partial(
        pl.kernel,
        mesh=mesh,
        out_type=jax.ShapeDtypeStruct((_NUM_ITEMS, 64), jnp.float32),
        scratch_types=[
            pltpu.VMEM((40, 128), jnp.int32),
            pltpu.VMEM((5136,), jnp.float32),
            pltpu.VMEM((640, 128), jnp.float32),
            pltpu.VMEM((128, 64), jnp.float32),
            pltpu.SemaphoreType.DMA,
        ],
    )(_hgather_body)
    return f(tab, cidx2d, cw_flat)


def kernel(Gu, Gi, feat_visual, feat_textual, W_visual, b_visual,
           W_textual, b_textual, w_imp, ui_edge_index, ui_values):
    fv = _rownorm(feat_visual)
    ft = _rownorm(feat_textual)
    pv = _proj_norm(fv, W_visual, b_visual)
    pt = _proj_norm(ft, W_textual, b_textual)

    vvR, ivR = _sim_topk(fv)
    vtR, itR = _sim_topk(ft)
    vvL, ivL = _sim_topk(pv)
    vtL, itL = _sim_topk(pt)

    cidx, cw, tab, hdiag = _prep(w_imp, Gi, ivL, vvL, itL, vtL,
                                 ivR, vvR, itR, vtR)

    hg = _hgather(tab, cidx.reshape(_NUM_ITEMS * 40 // 128, 128),
                  cw.reshape(-1))

    ego = jnp.concatenate([Gu, Gi], axis=0)
    half = ui_edge_index.shape[1] // 2
    row = ui_edge_index[0].astype(jnp.int32)
    col = ui_edge_index[1].astype(jnp.int32)
    row_local = jnp.concatenate([row[:half], row[half:] - _NUM_USERS])
    rowp2d = (row_local // 2).reshape(-1, 128)
    rowpar = (row_local % 2).astype(jnp.float32)
    col2d = col.reshape(-1, 128)
    zero_tab = jnp.zeros((_UPAIR, 128), jnp.float32)
    ego_dup = jnp.concatenate([ego, ego], axis=1)
    s1 = _ui_layer(ego_dup, rowp2d, col2d, ui_values, rowpar,
                   zero_tab).reshape(-1, 64)
    s1_dup = jnp.concatenate([s1, s1], axis=1)
    s2 = _ui_layer(s1_dup, rowp2d, col2d, ui_values, rowpar,
                   zero_tab).reshape(-1, 64)

    return _combine(ego, s1, s2, hg, hdiag)


# speedup vs baseline: 5.9857x; 1.0004x over previous
import functools

import jax
import jax.numpy as jnp
import numpy as np
from jax import lax
from jax.experimental import pallas as pl
from jax.experimental.pallas import tpu as pltpu
from jax.experimental.pallas import tpu_sc as plsc

_NUM_USERS = 16384
_NUM_ITEMS = 4096
_TOP_K = 10
_L_M = 0.7
_NEG = -3.4e38

_INTERPRET = False


def _rownorm_body(f_ref, o_ref):
    f = f_ref[...]
    nrm = jnp.sqrt(jnp.sum(f * f, axis=-1, keepdims=True))
    o_ref[...] = f / (nrm + 1e-12)


def _rownorm(f, block_rows=512):
    n, d = f.shape
    return pl.pallas_call(
        _rownorm_body,
        grid=(n // block_rows,),
        in_specs=[pl.BlockSpec((block_rows, d), lambda i: (i, 0))],
        out_specs=pl.BlockSpec((block_rows, d), lambda i: (i, 0)),
        out_shape=jax.ShapeDtypeStruct((n, d), jnp.float32),
        interpret=_INTERPRET,
    )(f)


def _proj_body(f_ref, w_ref, b_ref, o_ref):
    p = jax.lax.dot_general(f_ref[...], w_ref[...], (((1,), (0,)), ((), ())),
                            preferred_element_type=jnp.float32)
    p = p + b_ref[...]
    nrm = jnp.sqrt(jnp.sum(p * p, axis=-1, keepdims=True))
    o_ref[...] = p / (nrm + 1e-12)


def _proj_norm(f, w, b, block_rows=1024):
    n, d = f.shape
    k = w.shape[1]
    return pl.pallas_call(
        _proj_body,
        grid=(n // block_rows,),
        in_specs=[
            pl.BlockSpec((block_rows, d), lambda i: (i, 0)),
            pl.BlockSpec((d, k), lambda i: (0, 0)),
            pl.BlockSpec((1, k), lambda i: (0, 0)),
        ],
        out_specs=pl.BlockSpec((block_rows, k), lambda i: (i, 0)),
        out_shape=jax.ShapeDtypeStruct((n, k), jnp.float32),
        interpret=_INTERPRET,
    )(f, w, b.reshape(1, k))


def _simtopk_body(fi_ref, fj_ref, vals_ref, inds_ref, *, cblk, nj, k):
    j = pl.program_id(1)

    @pl.when(j == 0)
    def _():
        vals_ref[...] = jnp.full(vals_ref.shape, _NEG, jnp.float32)
        inds_ref[...] = jnp.zeros(inds_ref.shape, jnp.int32)

    scores = jax.lax.dot_general(
        fi_ref[...], fj_ref[...], (((1,), (1,)), ((), ())),
        preferred_element_type=jnp.float32)
    r = scores.shape[0]
    width = k + cblk
    col_iota = jax.lax.broadcasted_iota(jnp.int32, (r, cblk), 1)
    comb_v = jnp.concatenate([vals_ref[...], scores], axis=1)
    comb_i = jnp.concatenate([inds_ref[...], j * cblk + col_iota], axis=1)
    pos_iota = jax.lax.broadcasted_iota(jnp.int32, (r, width), 1)
    new_v = []
    new_i = []
    for _ in range(k):
        m = jnp.max(comb_v, axis=1, keepdims=True)
        hit = comb_v == m
        pos = jnp.min(jnp.where(hit, pos_iota, jnp.int32(2 ** 30)),
                      axis=1, keepdims=True)
        sel = pos_iota == pos
        gidx = jnp.sum(jnp.where(sel, comb_i, 0), axis=1, keepdims=True)
        new_v.append(m)
        new_i.append(gidx)
        comb_v = jnp.where(sel, _NEG, comb_v)
    vals_ref[...] = jnp.concatenate(new_v, axis=1)
    inds_ref[...] = jnp.concatenate(new_i, axis=1)


def _sim_topk(f, rblk=256, cblk=512, k=_TOP_K):
    n, d = f.shape
    ni, nj = n // rblk, n // cblk
    body = functools.partial(_simtopk_body, cblk=cblk, nj=nj, k=k)
    vals, inds = pl.pallas_call(
        body,
        grid=(ni, nj),
        in_specs=[
            pl.BlockSpec((rblk, d), lambda i, j: (i, 0)),
            pl.BlockSpec((cblk, d), lambda i, j: (j, 0)),
        ],
        out_specs=[
            pl.BlockSpec((rblk, k), lambda i, j: (i, 0)),
            pl.BlockSpec((rblk, k), lambda i, j: (i, 0)),
        ],
        out_shape=[
            jax.ShapeDtypeStruct((n, k), jnp.float32),
            jax.ShapeDtypeStruct((n, k), jnp.int32),
        ],
        interpret=_INTERPRET,
    )(f, f)
    return vals, inds


def _prep_body(wimp_ref, gi_ref,
               ivL_ref, vvL_ref, itL_ref, vtL_ref,
               ivR_ref, vvR_ref, itR_ref, vtR_ref,
               cidx_ref, cw_ref, tab_ref, hdiag_ref, *, n, rblk, k):
    wimp = wimp_ref[0, 0]
    wimp2 = wimp_ref[0, 1]
    mx = jnp.maximum(wimp, wimp2)
    e1 = jnp.exp(wimp - mx)
    e2 = jnp.exp(wimp2 - mx)
    sw0 = e1 / (e1 + e2)
    sw1 = e2 / (e1 + e2)

    row = (pl.program_id(0) * rblk
           + jax.lax.broadcasted_iota(jnp.int32, (rblk, 1), 0))

    def graph_deg(idx, val):
        valid = (idx != row) & (val != 0.0)
        return 1.0 + jnp.sum(valid.astype(jnp.float32), axis=1, keepdims=True)

    ivL, vvL = ivL_ref[...], vvL_ref[...]
    itL, vtL = itL_ref[...], vtL_ref[...]
    ivR, vvR = ivR_ref[...], vvR_ref[...]
    itR, vtR = itR_ref[...], vtR_ref[...]

    degLv = graph_deg(ivL, vvL)
    degLt = graph_deg(itL, vtL)
    degRv = graph_deg(ivR, vvR)
    degRt = graph_deg(itR, vtR)
    disLv = jax.lax.rsqrt(degLv)
    disLt = jax.lax.rsqrt(degLt)
    disRv = jax.lax.rsqrt(degRv)
    disRt = jax.lax.rsqrt(degRt)

    def union_deg(idx_a, val_a, idx_b, val_b):
        idx = jnp.concatenate([idx_a, idx_b], axis=1)
        val = jnp.concatenate([val_a, val_b], axis=1)
        valid = (idx != row) & (val != 0.0)
        cnt = jnp.zeros((rblk, 1), jnp.float32)
        for c in range(2 * k):
            ic = idx[:, c:c + 1]
            vc = valid[:, c:c + 1]
            dup = jnp.zeros((rblk, 1), jnp.bool_)
            for cc in range(c):
                dup = dup | (valid[:, cc:cc + 1] & (idx[:, cc:cc + 1] == ic))
            cnt = cnt + jnp.where(vc & ~dup, 1.0, 0.0)
        return 1.0 + cnt

    degL = union_deg(ivL, vvL, itL, vtL)
    degR = union_deg(ivR, vvR, itR, vtR)
    disL = jax.lax.rsqrt(degL)
    disR = jax.lax.rsqrt(degR)

    wL = 1.0 - _L_M
    wR = _L_M

    def group_w(outer_dis, outer_w, sw, inner_dis, idx, val):
        return outer_w * outer_dis * sw * inner_dis * val * (idx != row)

    cw_ref[...] = jnp.concatenate([
        group_w(disL, wL, sw0, disLv, ivL, vvL),
        group_w(disL, wL, sw1, disLt, itL, vtL),
        group_w(disR, wR, sw0, disRv, ivR, vvR),
        group_w(disR, wR, sw1, disRt, itR, vtR),
    ], axis=1)
    cidx_ref[...] = jnp.concatenate(
        [ivL, itL + n, ivR + 2 * n, itR + 3 * n], axis=1)

    gi = gi_ref[...]
    gi2 = jnp.concatenate([gi, gi], axis=1)
    tab_ref[0, :, :] = (disLv * disL) * gi2
    tab_ref[1, :, :] = (disLt * disL) * gi2
    tab_ref[2, :, :] = (disRv * disR) * gi2
    tab_ref[3, :, :] = (disRt * disR) * gi2
    hdiag_ref[...] = (wL / degL + wR / degR) * gi


def _prep(w_imp, gi, ivL, vvL, itL, vtL, ivR, vvR, itR, vtR, rblk=512):
    n, k = ivL.shape
    body = functools.partial(_prep_body, n=n, rblk=rblk, k=k)
    cidx, cw, tab, hdiag = pl.pallas_call(
        body,
        grid=(n // rblk,),
        in_specs=[
            pl.BlockSpec((1, 2), lambda i: (0, 0)),
            pl.BlockSpec((rblk, 64), lambda i: (i, 0)),
        ] + [pl.BlockSpec((rblk, k), lambda i: (i, 0))] * 8,
        out_specs=[
            pl.BlockSpec((rblk, 4 * k), lambda i: (i, 0)),
            pl.BlockSpec((rblk, 4 * k), lambda i: (i, 0)),
            pl.BlockSpec((4, rblk, 128), lambda i: (0, i, 0)),
            pl.BlockSpec((rblk, 64), lambda i: (i, 0)),
        ],
        out_shape=[
            jax.ShapeDtypeStruct((n, 4 * k), jnp.int32),
            jax.ShapeDtypeStruct((n, 4 * k), jnp.float32),
            jax.ShapeDtypeStruct((4, n, 128), jnp.float32),
            jax.ShapeDtypeStruct((n, 64), jnp.float32),
        ],
        interpret=_INTERPRET,
    )(w_imp.reshape(1, 2), gi, ivL, vvL, itL, vtL, ivR, vvR, itR, vtR)
    return cidx, cw, tab.reshape(4 * n, 128), hdiag


def _combine_body(e0_ref, s1_ref, s2_ref, hg_ref, hd_ref, gu_ref, gi_ref,
                  *, n_user_blocks):
    blk = pl.program_id(0)
    mean = (e0_ref[...] + s1_ref[...] + s2_ref[...]) * (1.0 / 3.0)

    @pl.when(blk < n_user_blocks)
    def _():
        gu_ref[...] = mean

    @pl.when(blk >= n_user_blocks)
    def _():
        h = hg_ref[...] + hd_ref[...]
        nrm = jnp.sqrt(jnp.sum(h * h, axis=-1, keepdims=True))
        gi_ref[...] = mean + h / (nrm + 1e-12)


def _combine(e0, s1, s2, hg, hd):
    blk = 2048
    nub = _NUM_USERS // blk
    nib = _NUM_ITEMS // blk
    kdim = 64

    body = functools.partial(_combine_body, n_user_blocks=nub)
    gu, gi = pl.pallas_call(
        body,
        grid=(nub + nib,),
        in_specs=[
            pl.BlockSpec((blk, kdim), lambda i: (i, 0)),
            pl.BlockSpec((blk, kdim), lambda i: (i, 0)),
            pl.BlockSpec((blk, kdim), lambda i: (i, 0)),
            pl.BlockSpec((blk, kdim), lambda i: (jnp.maximum(i - nub, 0), 0)),
            pl.BlockSpec((blk, kdim), lambda i: (jnp.maximum(i - nub, 0), 0)),
        ],
        out_specs=[
            pl.BlockSpec((blk, kdim), lambda i: (jnp.minimum(i, nub - 1), 0)),
            pl.BlockSpec((blk, kdim), lambda i: (jnp.maximum(i - nub, 0), 0)),
        ],
        out_shape=[
            jax.ShapeDtypeStruct((_NUM_USERS, kdim), jnp.float32),
            jax.ShapeDtypeStruct((_NUM_ITEMS, kdim), jnp.float32),
        ],
        interpret=_INTERPRET,
    )(e0, s1, s2, hg, hd)
    return gu, gi


_EPC = (1048576 // 2) // 16
_SCHUNK = 256
_NV = _SCHUNK // 128
_UPAIR = _NUM_USERS // 2
_IPAIR = _NUM_ITEMS // 2


def _ui_layer_body(emb_hbm, rowp2d_hbm, col2d_hbm, val_hbm, par_hbm,
                   zero_hbm, out_hbm, acc, rowb, colb, valb, parb, gbuf,
                   sem):
    c = lax.axis_index("c")
    s = lax.axis_index("s")
    npair = jnp.where(c == 0, _UPAIR, _IPAIR)
    zrows = npair // 16
    step = _IPAIR // 16
    sz = pl.multiple_of(s * zrows, step)
    pltpu.sync_copy(zero_hbm.at[pl.ds(sz, step)], acc.at[pl.ds(sz, step)])

    @pl.when(c == 0)
    def _():
        zrows_u = _UPAIR // 16
        for t in range(_UPAIR // _IPAIR - 1):
            off = (t + 1) * step
            b = pl.multiple_of(s * zrows_u + off, step)
            pltpu.sync_copy(zero_hbm.at[pl.ds(b, step)],
                            acc.at[pl.ds(b, step)])

    plsc.subcore_barrier()

    vec_base = (c * 16 + s) * (_EPC // 128)

    def chunk_body(g, carry):
        vb = pl.multiple_of(vec_base + g * _NV, _NV)
        pltpu.sync_copy(rowp2d_hbm.at[pl.ds(vb, _NV)], rowb)
        pltpu.sync_copy(col2d_hbm.at[pl.ds(vb, _NV)], colb)
        pltpu.sync_copy(val_hbm.at[pl.ds(vb * 128, _SCHUNK)], valb)
        pltpu.sync_copy(par_hbm.at[pl.ds(vb * 128, _SCHUNK)], parb)
        cps = [
            pltpu.async_copy(emb_hbm.at[colb.at[j]],
                             gbuf.at[pl.ds(j * 128, 128)], sem)
            for j in range(_NV)
        ]
        for cp in cps:
            cp.wait()

        def mul_body(g2, _):
            v16 = valb[pl.ds(g2 * 16, 16)]
            p16 = parb[pl.ds(g2 * 16, 16)]
            for l in range(16):
                w = v16[l]
                p = p16[l]
                w1 = w * p
                w0 = w - w1
                e = g2 * 16 + l
                for q in range(4):
                    gbuf[e, pl.ds(q * 16, 16)] = (
                        gbuf[e, pl.ds(q * 16, 16)] * w0)
                    gbuf[e, pl.ds(64 + q * 16, 16)] = (
                        gbuf[e, pl.ds(64 + q * 16, 16)] * w1)
            return _

        lax.fori_loop(0, _SCHUNK // 16, mul_body, None)
        for j in range(_NV):
            pltpu.sync_copy(gbuf.at[pl.ds(j * 128, 128)],
                            acc.at[rowb.at[j]], add=True)
        return carry

    lax.fori_loop(0, _EPC // _SCHUNK, chunk_body, None)
    plsc.subcore_barrier()

    so = pl.multiple_of(s * zrows, step)
    obase = pl.multiple_of(jnp.where(c == 0, 0, _UPAIR) + s * zrows, step)
    pltpu.sync_copy(acc.at[pl.ds(so, step)], out_hbm.at[pl.ds(obase, step)])

    @pl.when(c == 0)
    def _():
        zrows_u = _UPAIR // 16
        for t in range(_UPAIR // _IPAIR - 1):
            off = (t + 1) * step
            b = pl.multiple_of(s * zrows_u + off, step)
            pltpu.sync_copy(acc.at[pl.ds(b, step)],
                            out_hbm.at[pl.ds(b, step)])


def _ui_layer(emb_dup, rowp2d, col2d, val, par, zero_tab):
    mesh = plsc.VectorSubcoreMesh(core_axis_name="c", subcore_axis_name="s")
    f = functools.partial(
        pl.kernel,
        mesh=mesh,
        out_type=jax.ShapeDtypeStruct((_UPAIR + _IPAIR, 128), jnp.float32),
        scratch_types=[
            pltpu.VMEM_SHARED((_UPAIR, 128), jnp.float32),
            pltpu.VMEM((_NV, 128), jnp.int32),
            pltpu.VMEM((_NV, 128), jnp.int32),
            pltpu.VMEM((_SCHUNK,), jnp.float32),
            pltpu.VMEM((_SCHUNK,), jnp.float32),
            pltpu.VMEM((_SCHUNK, 128), jnp.float32),
            pltpu.SemaphoreType.DMA,
        ],
    )(_ui_layer_body)
    return f(emb_dup, rowp2d, col2d, val, par, zero_tab)


def _hgather_body(tab_hbm, cidx2d_hbm, cw_hbm, out_hbm,
                  idxb, wb, gbuf, ob, sem):
    c = lax.axis_index("c")
    s = lax.axis_index("s")
    t = c * 16 + s
    pltpu.sync_copy(cidx2d_hbm.at[pl.ds(pl.multiple_of(t * 40, 40), 40)],
                    idxb)
    pltpu.sync_copy(cw_hbm.at[pl.ds(pl.multiple_of(t * 5120, 5120), 5120)],
                    wb.at[pl.ds(0, 5120)])
    for g in range(8):
        cps = [
            pltpu.async_copy(tab_hbm.at[idxb.at[g * 5 + j]],
                             gbuf.at[pl.ds(j * 128, 128)], sem)
            for j in range(5)
        ]
        for cp in cps:
            cp.wait()

        def row_body(r, _):
            zero = jnp.zeros((16,), jnp.float32)
            accs = [zero] * 4
            wbase = g * 640 + r * 40
            for blk in range(3):
                v16 = wb[pl.ds(wbase + blk * 16, 16)]
                nl = 16 if blk < 2 else 8
                for l in range(nl):
                    w = v16[l]
                    fi = r * 40 + blk * 16 + l
                    for q in range(4):
                        accs[q] = accs[q] + w * gbuf[fi, pl.ds(q * 16, 16)]
            for q in range(4):
                ob[g * 16 + r, pl.ds(q * 16, 16)] = accs[q]
            return _

        lax.fori_loop(0, 16, row_body, None)
    pltpu.sync_copy(ob, out_hbm.at[pl.ds(pl.multiple_of(t * 128, 128), 128)])


def _hgather(tab, cidx2d, cw_flat):
    mesh = plsc.VectorSubcoreMesh(core_axis_name="c", subcore_axis_name="s")
    f = functools.partial(
        pl.kernel,
        mesh=mesh,
        out_type=jax.ShapeDtypeStruct((_NUM_ITEMS, 64), jnp.float32),
        scratch_types=[
            pltpu.VMEM((40, 128), jnp.int32),
            pltpu.VMEM((5136,), jnp.float32),
            pltpu.VMEM((640, 128), jnp.float32),
            pltpu.VMEM((128, 64), jnp.float32),
            pltpu.SemaphoreType.DMA,
        ],
    )(_hgather_body)
    return f(tab, cidx2d, cw_flat)


def kernel(Gu, Gi, feat_visual, feat_textual, W_visual, b_visual,
           W_textual, b_textual, w_imp, ui_edge_index, ui_values):
    ego = jnp.concatenate([Gu, Gi], axis=0)
    half = ui_edge_index.shape[1] // 2
    row = ui_edge_index[0].astype(jnp.int32)
    col = ui_edge_index[1].astype(jnp.int32)
    row_local = jnp.concatenate([row[:half], row[half:] - _NUM_USERS])
    rowp2d = (row_local // 2).reshape(-1, 128)
    rowpar = (row_local % 2).astype(jnp.float32)
    col2d = col.reshape(-1, 128)
    zero_tab = jnp.zeros((_UPAIR, 128), jnp.float32)
    ego_dup = jnp.concatenate([ego, ego], axis=1)
    s1 = _ui_layer(ego_dup, rowp2d, col2d, ui_values, rowpar,
                   zero_tab).reshape(-1, 64)

    fv = _rownorm(feat_visual)
    ft = _rownorm(feat_textual)
    pv = _proj_norm(fv, W_visual, b_visual)
    pt = _proj_norm(ft, W_textual, b_textual)

    vvR, ivR = _sim_topk(fv)

    s1_dup = jnp.concatenate([s1, s1], axis=1)
    s2 = _ui_layer(s1_dup, rowp2d, col2d, ui_values, rowpar,
                   zero_tab).reshape(-1, 64)

    vtR, itR = _sim_topk(ft)
    vvL, ivL = _sim_topk(pv)
    vtL, itL = _sim_topk(pt)

    cidx, cw, tab, hdiag = _prep(w_imp, Gi, ivL, vvL, itL, vtL,
                                 ivR, vvR, itR, vtR)

    hg = _hgather(tab, cidx.reshape(_NUM_ITEMS * 40 // 128, 128),
                  cw.reshape(-1))

    return _combine(ego, s1, s2, hg, hdiag)


# speedup vs baseline: 6.0219x; 1.0060x over previous
import functools

import jax
import jax.numpy as jnp
import numpy as np
from jax import lax
from jax.experimental import pallas as pl
from jax.experimental.pallas import tpu as pltpu
from jax.experimental.pallas import tpu_sc as plsc

_NUM_USERS = 16384
_NUM_ITEMS = 4096
_TOP_K = 10
_L_M = 0.7
_NEG = -3.4e38

_INTERPRET = False


def _rownorm_body(f_ref, o_ref):
    f = f_ref[...]
    nrm = jnp.sqrt(jnp.sum(f * f, axis=-1, keepdims=True))
    o_ref[...] = f / (nrm + 1e-12)


def _rownorm(f, block_rows=512):
    n, d = f.shape
    return pl.pallas_call(
        _rownorm_body,
        grid=(n // block_rows,),
        in_specs=[pl.BlockSpec((block_rows, d), lambda i: (i, 0))],
        out_specs=pl.BlockSpec((block_rows, d), lambda i: (i, 0)),
        out_shape=jax.ShapeDtypeStruct((n, d), jnp.float32),
        interpret=_INTERPRET,
    )(f)


def _proj_body(f_ref, w_ref, b_ref, o_ref):
    p = jax.lax.dot_general(f_ref[...], w_ref[...], (((1,), (0,)), ((), ())),
                            preferred_element_type=jnp.float32)
    p = p + b_ref[...]
    nrm = jnp.sqrt(jnp.sum(p * p, axis=-1, keepdims=True))
    o_ref[...] = p / (nrm + 1e-12)


def _proj_norm(f, w, b, block_rows=1024):
    n, d = f.shape
    k = w.shape[1]
    return pl.pallas_call(
        _proj_body,
        grid=(n // block_rows,),
        in_specs=[
            pl.BlockSpec((block_rows, d), lambda i: (i, 0)),
            pl.BlockSpec((d, k), lambda i: (0, 0)),
            pl.BlockSpec((1, k), lambda i: (0, 0)),
        ],
        out_specs=pl.BlockSpec((block_rows, k), lambda i: (i, 0)),
        out_shape=jax.ShapeDtypeStruct((n, k), jnp.float32),
        interpret=_INTERPRET,
    )(f, w, b.reshape(1, k))


def _simtopk_body(fi_ref, fj_ref, vals_ref, inds_ref, *, cblk, nj, k):
    j = pl.program_id(1)

    @pl.when(j == 0)
    def _():
        vals_ref[...] = jnp.full(vals_ref.shape, _NEG, jnp.float32)
        inds_ref[...] = jnp.zeros(inds_ref.shape, jnp.int32)

    scores = jax.lax.dot_general(
        fi_ref[...], fj_ref[...], (((1,), (1,)), ((), ())),
        preferred_element_type=jnp.float32)
    r = scores.shape[0]
    width = k + cblk
    col_iota = jax.lax.broadcasted_iota(jnp.int32, (r, cblk), 1)
    comb_v = jnp.concatenate([vals_ref[...], scores], axis=1)
    comb_i = jnp.concatenate([inds_ref[...], j * cblk + col_iota], axis=1)
    pos_iota = jax.lax.broadcasted_iota(jnp.int32, (r, width), 1)
    new_v = []
    new_i = []
    for _ in range(k):
        m = jnp.max(comb_v, axis=1, keepdims=True)
        hit = comb_v == m
        pos = jnp.min(jnp.where(hit, pos_iota, jnp.int32(2 ** 30)),
                      axis=1, keepdims=True)
        sel = pos_iota == pos
        gidx = jnp.sum(jnp.where(sel, comb_i, 0), axis=1, keepdims=True)
        new_v.append(m)
        new_i.append(gidx)
        comb_v = jnp.where(sel, _NEG, comb_v)
    vals_ref[...] = jnp.concatenate(new_v, axis=1)
    inds_ref[...] = jnp.concatenate(new_i, axis=1)


def _sim_topk(f, rblk=256, cblk=512, k=_TOP_K):
    n, d = f.shape
    ni, nj = n // rblk, n // cblk
    body = functools.partial(_simtopk_body, cblk=cblk, nj=nj, k=k)
    vals, inds = pl.pallas_call(
        body,
        grid=(ni, nj),
        in_specs=[
            pl.BlockSpec((rblk, d), lambda i, j: (i, 0)),
            pl.BlockSpec((cblk, d), lambda i, j: (j, 0)),
        ],
        out_specs=[
            pl.BlockSpec((rblk, k), lambda i, j: (i, 0)),
            pl.BlockSpec((rblk, k), lambda i, j: (i, 0)),
        ],
        out_shape=[
            jax.ShapeDtypeStruct((n, k), jnp.float32),
            jax.ShapeDtypeStruct((n, k), jnp.int32),
        ],
        interpret=_INTERPRET,
    )(f, f)
    return vals, inds


def _prep_body(wimp_ref, gi_ref,
               ivL_ref, vvL_ref, itL_ref, vtL_ref,
               ivR_ref, vvR_ref, itR_ref, vtR_ref,
               cidx_ref, cw_ref, tab_ref, hdiag_ref, *, n, rblk, k):
    wimp = wimp_ref[0, 0]
    wimp2 = wimp_ref[0, 1]
    mx = jnp.maximum(wimp, wimp2)
    e1 = jnp.exp(wimp - mx)
    e2 = jnp.exp(wimp2 - mx)
    sw0 = e1 / (e1 + e2)
    sw1 = e2 / (e1 + e2)

    row = (pl.program_id(0) * rblk
           + jax.lax.broadcasted_iota(jnp.int32, (rblk, 1), 0))

    def graph_deg(idx, val):
        valid = (idx != row) & (val != 0.0)
        return 1.0 + jnp.sum(valid.astype(jnp.float32), axis=1, keepdims=True)

    ivL, vvL = ivL_ref[...], vvL_ref[...]
    itL, vtL = itL_ref[...], vtL_ref[...]
    ivR, vvR = ivR_ref[...], vvR_ref[...]
    itR, vtR = itR_ref[...], vtR_ref[...]

    degLv = graph_deg(ivL, vvL)
    degLt = graph_deg(itL, vtL)
    degRv = graph_deg(ivR, vvR)
    degRt = graph_deg(itR, vtR)
    disLv = jax.lax.rsqrt(degLv)
    disLt = jax.lax.rsqrt(degLt)
    disRv = jax.lax.rsqrt(degRv)
    disRt = jax.lax.rsqrt(degRt)

    def union_deg(idx_a, val_a, idx_b, val_b):
        idx = jnp.concatenate([idx_a, idx_b], axis=1)
        val = jnp.concatenate([val_a, val_b], axis=1)
        valid = (idx != row) & (val != 0.0)
        cnt = jnp.zeros((rblk, 1), jnp.float32)
        for c in range(2 * k):
            ic = idx[:, c:c + 1]
            vc = valid[:, c:c + 1]
            dup = jnp.zeros((rblk, 1), jnp.bool_)
            for cc in range(c):
                dup = dup | (valid[:, cc:cc + 1] & (idx[:, cc:cc + 1] == ic))
            cnt = cnt + jnp.where(vc & ~dup, 1.0, 0.0)
        return 1.0 + cnt

    degL = union_deg(ivL, vvL, itL, vtL)
    degR = union_deg(ivR, vvR, itR, vtR)
    disL = jax.lax.rsqrt(degL)
    disR = jax.lax.rsqrt(degR)

    wL = 1.0 - _L_M
    wR = _L_M

    def group_w(outer_dis, outer_w, sw, inner_dis, idx, val):
        return outer_w * outer_dis * sw * inner_dis * val * (idx != row)

    cw_ref[...] = jnp.concatenate([
        group_w(disL, wL, sw0, disLv, ivL, vvL),
        group_w(disL, wL, sw1, disLt, itL, vtL),
        group_w(disR, wR, sw0, disRv, ivR, vvR),
        group_w(disR, wR, sw1, disRt, itR, vtR),
    ], axis=1)
    cidx_ref[...] = jnp.concatenate(
        [ivL, itL + n, ivR + 2 * n, itR + 3 * n], axis=1)

    gi = gi_ref[...]
    gi2 = jnp.concatenate([gi, gi], axis=1)
    tab_ref[0, :, :] = (disLv * disL) * gi2
    tab_ref[1, :, :] = (disLt * disL) * gi2
    tab_ref[2, :, :] = (disRv * disR) * gi2
    tab_ref[3, :, :] = (disRt * disR) * gi2
    hdiag_ref[...] = (wL / degL + wR / degR) * gi


def _prep(w_imp, gi, ivL, vvL, itL, vtL, ivR, vvR, itR, vtR, rblk=512):
    n, k = ivL.shape
    body = functools.partial(_prep_body, n=n, rblk=rblk, k=k)
    cidx, cw, tab, hdiag = pl.pallas_call(
        body,
        grid=(n // rblk,),
        in_specs=[
            pl.BlockSpec((1, 2), lambda i: (0, 0)),
            pl.BlockSpec((rblk, 64), lambda i: (i, 0)),
        ] + [pl.BlockSpec((rblk, k), lambda i: (i, 0))] * 8,
        out_specs=[
            pl.BlockSpec((rblk, 4 * k), lambda i: (i, 0)),
            pl.BlockSpec((rblk, 4 * k), lambda i: (i, 0)),
            pl.BlockSpec((4, rblk, 128), lambda i: (0, i, 0)),
            pl.BlockSpec((rblk, 64), lambda i: (i, 0)),
        ],
        out_shape=[
            jax.ShapeDtypeStruct((n, 4 * k), jnp.int32),
            jax.ShapeDtypeStruct((n, 4 * k), jnp.float32),
            jax.ShapeDtypeStruct((4, n, 128), jnp.float32),
            jax.ShapeDtypeStruct((n, 64), jnp.float32),
        ],
        interpret=_INTERPRET,
    )(w_imp.reshape(1, 2), gi, ivL, vvL, itL, vtL, ivR, vvR, itR, vtR)
    return cidx, cw, tab.reshape(4 * n, 128), hdiag


def _combine_body(e0_ref, s1_ref, s2_ref, hg_ref, hd_ref, gu_ref, gi_ref,
                  *, n_user_blocks):
    blk = pl.program_id(0)
    mean = (e0_ref[...] + s1_ref[...] + s2_ref[...]) * (1.0 / 3.0)

    @pl.when(blk < n_user_blocks)
    def _():
        gu_ref[...] = mean

    @pl.when(blk >= n_user_blocks)
    def _():
        h = hg_ref[...] + hd_ref[...]
        nrm = jnp.sqrt(jnp.sum(h * h, axis=-1, keepdims=True))
        gi_ref[...] = mean + h / (nrm + 1e-12)


def _combine(e0, s1, s2, hg, hd):
    blk = 2048
    nub = _NUM_USERS // blk
    nib = _NUM_ITEMS // blk
    kdim = 64

    body = functools.partial(_combine_body, n_user_blocks=nub)
    gu, gi = pl.pallas_call(
        body,
        grid=(nub + nib,),
        in_specs=[
            pl.BlockSpec((blk, kdim), lambda i: (i, 0)),
            pl.BlockSpec((blk, kdim), lambda i: (i, 0)),
            pl.BlockSpec((blk, kdim), lambda i: (i, 0)),
            pl.BlockSpec((blk, kdim), lambda i: (jnp.maximum(i - nub, 0), 0)),
            pl.BlockSpec((blk, kdim), lambda i: (jnp.maximum(i - nub, 0), 0)),
        ],
        out_specs=[
            pl.BlockSpec((blk, kdim), lambda i: (jnp.minimum(i, nub - 1), 0)),
            pl.BlockSpec((blk, kdim), lambda i: (jnp.maximum(i - nub, 0), 0)),
        ],
        out_shape=[
            jax.ShapeDtypeStruct((_NUM_USERS, kdim), jnp.float32),
            jax.ShapeDtypeStruct((_NUM_ITEMS, kdim), jnp.float32),
        ],
        interpret=_INTERPRET,
    )(e0, s1, s2, hg, hd)
    return gu, gi


_EPC = (1048576 // 2) // 16
_SCHUNK = 128
_UPAIR = _NUM_USERS // 2
_IPAIR = _NUM_ITEMS // 2


def _ui_layer_body(emb_hbm, rowp2d_hbm, col2d_hbm, val0_hbm, val1_hbm,
                   zero_hbm, out_hbm, acc,
                   rowbA, colbA, v0A, v1A, gbufA,
                   rowbB, colbB, v0B, v1B, gbufB, semA, semB):
    c = lax.axis_index("c")
    s = lax.axis_index("s")
    npair = jnp.where(c == 0, _UPAIR, _IPAIR)
    zrows = npair // 16
    step = _IPAIR // 16
    sz = pl.multiple_of(s * zrows, step)
    pltpu.sync_copy(zero_hbm.at[pl.ds(sz, step)], acc.at[pl.ds(sz, step)])

    @pl.when(c == 0)
    def _():
        zrows_u = _UPAIR // 16
        for t in range(_UPAIR // _IPAIR - 1):
            off = (t + 1) * step
            b = pl.multiple_of(s * zrows_u + off, step)
            pltpu.sync_copy(zero_hbm.at[pl.ds(b, step)],
                            acc.at[pl.ds(b, step)])

    plsc.subcore_barrier()

    vec_base = (c * 16 + s) * (_EPC // 128)

    def load_and_issue(ci, rowb, colb, v0b, v1b, gbuf, sem):
        vb = pl.multiple_of(vec_base + ci, 1)
        pltpu.sync_copy(rowp2d_hbm.at[pl.ds(vb, 1)], rowb)
        pltpu.sync_copy(col2d_hbm.at[pl.ds(vb, 1)], colb)
        pltpu.sync_copy(val0_hbm.at[pl.ds(vb * 128, _SCHUNK)], v0b)
        pltpu.sync_copy(val1_hbm.at[pl.ds(vb * 128, _SCHUNK)], v1b)
        return pltpu.async_copy(emb_hbm.at[colb.at[0]], gbuf, sem)

    def process(cp, rowb, v0b, v1b, gbuf):
        cp.wait()

        def mul_body(g2, _):
            w0_16 = v0b[pl.ds(g2 * 16, 16)]
            w1_16 = v1b[pl.ds(g2 * 16, 16)]
            for l in range(16):
                w0 = w0_16[l]
                w1 = w1_16[l]
                e = g2 * 16 + l
                for q in range(4):
                    gbuf[e, pl.ds(q * 16, 16)] = (
                        gbuf[e, pl.ds(q * 16, 16)] * w0)
                    gbuf[e, pl.ds(64 + q * 16, 16)] = (
                        gbuf[e, pl.ds(64 + q * 16, 16)] * w1)
            return _

        lax.fori_loop(0, _SCHUNK // 16, mul_body, None)
        pltpu.sync_copy(gbuf, acc.at[rowb.at[0]], add=True)

    def pair_body(gp, carry):
        cpA = load_and_issue(2 * gp, rowbA, colbA, v0A, v1A, gbufA, semA)
        cpB = load_and_issue(2 * gp + 1, rowbB, colbB, v0B, v1B, gbufB,
                             semB)
        process(cpA, rowbA, v0A, v1A, gbufA)
        process(cpB, rowbB, v0B, v1B, gbufB)
        return carry

    lax.fori_loop(0, _EPC // (2 * _SCHUNK), pair_body, None)
    plsc.subcore_barrier()

    so = pl.multiple_of(s * zrows, step)
    obase = pl.multiple_of(jnp.where(c == 0, 0, _UPAIR) + s * zrows, step)
    pltpu.sync_copy(acc.at[pl.ds(so, step)], out_hbm.at[pl.ds(obase, step)])

    @pl.when(c == 0)
    def _():
        zrows_u = _UPAIR // 16
        for t in range(_UPAIR // _IPAIR - 1):
            off = (t + 1) * step
            b = pl.multiple_of(s * zrows_u + off, step)
            pltpu.sync_copy(acc.at[pl.ds(b, step)],
                            out_hbm.at[pl.ds(b, step)])


def _ui_layer(emb_dup, rowp2d, col2d, val0, val1, zero_tab):
    mesh = plsc.VectorSubcoreMesh(core_axis_name="c", subcore_axis_name="s")
    buf_types = [
        pltpu.VMEM((1, 128), jnp.int32),
        pltpu.VMEM((1, 128), jnp.int32),
        pltpu.VMEM((_SCHUNK,), jnp.float32),
        pltpu.VMEM((_SCHUNK,), jnp.float32),
        pltpu.VMEM((_SCHUNK, 128), jnp.float32),
    ]
    f = functools.partial(
        pl.kernel,
        mesh=mesh,
        out_type=jax.ShapeDtypeStruct((_UPAIR + _IPAIR, 128), jnp.float32),
        scratch_types=[
            pltpu.VMEM_SHARED((_UPAIR, 128), jnp.float32),
        ] + buf_types + buf_types + [
            pltpu.SemaphoreType.DMA,
            pltpu.SemaphoreType.DMA,
        ],
    )(_ui_layer_body)
    return f(emb_dup, rowp2d, col2d, val0, val1, zero_tab)


def _hgather_body(tab_hbm, cidx2d_hbm, cw_hbm, out_hbm,
                  idxb, wb, gbuf, ob, sem):
    c = lax.axis_index("c")
    s = lax.axis_index("s")
    t = c * 16 + s
    pltpu.sync_copy(cidx2d_hbm.at[pl.ds(pl.multiple_of(t * 40, 40), 40)],
                    idxb)
    pltpu.sync_copy(cw_hbm.at[pl.ds(pl.multiple_of(t * 5120, 5120), 5120)],
                    wb.at[pl.ds(0, 5120)])
    for g in range(8):
        cps = [
            pltpu.async_copy(tab_hbm.at[idxb.at[g * 5 + j]],
                             gbuf.at[pl.ds(j * 128, 128)], sem)
            for j in range(5)
        ]
        for cp in cps:
            cp.wait()

        def row_body(r, _):
            zero = jnp.zeros((16,), jnp.float32)
            accs = [zero] * 4
            wbase = g * 640 + r * 40
            for blk in range(3):
                v16 = wb[pl.ds(wbase + blk * 16, 16)]
                nl = 16 if blk < 2 else 8
                for l in range(nl):
                    w = v16[l]
                    fi = r * 40 + blk * 16 + l
                    for q in range(4):
                        accs[q] = accs[q] + w * gbuf[fi, pl.ds(q * 16, 16)]
            for q in range(4):
                ob[g * 16 + r, pl.ds(q * 16, 16)] = accs[q]
            return _

        lax.fori_loop(0, 16, row_body, None)
    pltpu.sync_copy(ob, out_hbm.at[pl.ds(pl.multiple_of(t * 128, 128), 128)])


def _hgather(tab, cidx2d, cw_flat):
    mesh = plsc.VectorSubcoreMesh(core_axis_name="c", subcore_axis_name="s")
    f = functools.partial(
        pl.kernel,
        mesh=mesh,
        out_type=jax.ShapeDtypeStruct((_NUM_ITEMS, 64), jnp.float32),
        scratch_types=[
            pltpu.VMEM((40, 128), jnp.int32),
            pltpu.VMEM((5136,), jnp.float32),
            pltpu.VMEM((640, 128), jnp.float32),
            pltpu.VMEM((128, 64), jnp.float32),
            pltpu.SemaphoreType.DMA,
        ],
    )(_hgather_body)
    return f(tab, cidx2d, cw_flat)


def kernel(Gu, Gi, feat_visual, feat_textual, W_visual, b_visual,
           W_textual, b_textual, w_imp, ui_edge_index, ui_values):
    ego = jnp.concatenate([Gu, Gi], axis=0)
    half = ui_edge_index.shape[1] // 2
    row = ui_edge_index[0].astype(jnp.int32)
    col = ui_edge_index[1].astype(jnp.int32)
    row_local = jnp.concatenate([row[:half], row[half:] - _NUM_USERS])
    rowp2d = (row_local // 2).reshape(-1, 128)
    rowpar = (row_local % 2).astype(jnp.float32)
    vals = ui_values.astype(jnp.float32)
    val1 = vals * rowpar
    val0 = vals - val1
    col2d = col.reshape(-1, 128)
    zero_tab = jnp.zeros((_UPAIR, 128), jnp.float32)
    ego_dup = jnp.concatenate([ego, ego], axis=1)
    s1 = _ui_layer(ego_dup, rowp2d, col2d, val0, val1,
                   zero_tab).reshape(-1, 64)

    fv = _rownorm(feat_visual)
    ft = _rownorm(feat_textual)
    pv = _proj_norm(fv, W_visual, b_visual)
    pt = _proj_norm(ft, W_textual, b_textual)

    vvR, ivR = _sim_topk(fv)

    s1_dup = jnp.concatenate([s1, s1], axis=1)
    s2 = _ui_layer(s1_dup, rowp2d, col2d, val0, val1,
                   zero_tab).reshape(-1, 64)

    vtR, itR = _sim_topk(ft)
    vvL, ivL = _sim_topk(pv)
    vtL, itL = _sim_topk(pt)

    cidx, cw, tab, hdiag = _prep(w_imp, Gi, ivL, vvL, itL, vtL,
                                 ivR, vvR, itR, vtR)

    hg = _hgather(tab, cidx.reshape(_NUM_ITEMS * 40 // 128, 128),
                  cw.reshape(-1))

    return _combine(ego, s1, s2, hg, hdiag)
